# Initial kernel scaffold; baseline (speedup 1.0000x reference)
import jax
import jax.numpy as jnp
from jax.experimental import pallas as pl


def kernel(word_emb, pos_emb, fc_out_w, fc_out_b, l0_sa_wq, l0_sa_wk, l0_sa_wv, l0_sa_wo, l0_sa_bo, l0_norm_g, l0_norm_b, l0_ca_wq, l0_ca_wk, l0_ca_wv, l0_ca_wo, l0_ca_bo, l0_norm1_g, l0_norm1_b, l0_norm2_g, l0_norm2_b, l0_ff_w1, l0_ff_b1, l0_ff_w2, l0_ff_b2, l1_sa_wq, l1_sa_wk, l1_sa_wv, l1_sa_wo, l1_sa_bo, l1_norm_g, l1_norm_b, l1_ca_wq, l1_ca_wk, l1_ca_wv, l1_ca_wo, l1_ca_bo, l1_norm1_g, l1_norm1_b, l1_norm2_g, l1_norm2_b, l1_ff_w1, l1_ff_b1, l1_ff_w2, l1_ff_b2, tokens, enc_out, src_mask):
    raise NotImplementedError("write your pallas kernel here")


# speedup vs baseline: 1.3141x; 1.3141x over previous
import jax
import jax.numpy as jnp
from jax import lax
from jax.experimental import pallas as pl
from jax.experimental.pallas import tpu as pltpu

_HEADS = 8
_NEG = -1e30


def _layernorm(y, g_ref, b_ref):
    mean = jnp.mean(y, axis=-1, keepdims=True)
    var = jnp.mean((y - mean) ** 2, axis=-1, keepdims=True)
    return (((y - mean) * lax.rsqrt(var + 1e-5)) * g_ref[...]
            + b_ref[...]).astype(jnp.bfloat16)


def _attn_ln(xq, K, V, wq_ref, wo_ref, bo_ref, g_ref, b_ref, keep):
    Lq, E = xq.shape
    Lk = K.shape[0]
    H = _HEADS
    D = E // H

    Q = jnp.dot(xq, wq_ref[...], preferred_element_type=jnp.float32)
    Qh = Q.reshape(Lq, H, D).astype(jnp.bfloat16)
    Kh = K.reshape(Lk, H, D)
    Vh = V.reshape(Lk, H, D)

    s = jnp.einsum('qhd,khd->hqk', Qh, Kh,
                   preferred_element_type=jnp.float32)
    if keep is None:
        row = lax.broadcasted_iota(jnp.int32, (Lq, Lk), 0)
        col = lax.broadcasted_iota(jnp.int32, (Lq, Lk), 1)
        s = jnp.where((row >= col)[None, :, :], s, _NEG)
    else:
        s = jnp.where(keep[None, None, :], s, _NEG)

    m = jnp.max(s, axis=-1, keepdims=True)
    p = jnp.exp(s - m)
    linv = 1.0 / jnp.sum(p, axis=-1, keepdims=True)
    ctx = jnp.einsum('hqk,khd->hqd', p.astype(jnp.bfloat16), Vh,
                     preferred_element_type=jnp.float32)
    ctx = ctx * linv
    ctx = jnp.transpose(ctx, (1, 0, 2)).reshape(Lq, E).astype(jnp.bfloat16)

    attn = jnp.dot(ctx, wo_ref[...],
                   preferred_element_type=jnp.float32) + bo_ref[...]
    return _layernorm(attn + xq.astype(jnp.float32), g_ref, b_ref)


def _decoder_body_kernel(x_ref, enc_ref, keep_ref, *refs):
    out_ref = refs[-1]
    wrs = refs[:-1]
    num_layers = len(wrs) // 20

    x = x_ref[...]
    enc = enc_ref[...]
    keep = keep_ref[0, :] != 0.0

    for li in range(num_layers):
        (sa_wq, sa_wk, sa_wv, sa_wo, sa_bo, ng, nb,
         ca_wq, ca_wk, ca_wv, ca_wo, ca_bo, n1g, n1b,
         w1, b1, w2, b2, n2g, n2b) = wrs[20 * li:20 * (li + 1)]

        K = jnp.dot(x, sa_wk[...],
                    preferred_element_type=jnp.float32).astype(jnp.bfloat16)
        V = jnp.dot(x, sa_wv[...],
                    preferred_element_type=jnp.float32).astype(jnp.bfloat16)
        q = _attn_ln(x, K, V, sa_wq, sa_wo, sa_bo, ng, nb, None)

        K2 = jnp.dot(enc, ca_wk[...],
                     preferred_element_type=jnp.float32).astype(jnp.bfloat16)
        V2 = jnp.dot(enc, ca_wv[...],
                     preferred_element_type=jnp.float32).astype(jnp.bfloat16)
        x1 = _attn_ln(q, K2, V2, ca_wq, ca_wo, ca_bo, n1g, n1b, keep)

        h = jnp.dot(x1, w1[...],
                    preferred_element_type=jnp.float32) + b1[...]
        h = jnp.maximum(h, 0.0).astype(jnp.bfloat16)
        y = (jnp.dot(h, w2[...], preferred_element_type=jnp.float32)
             + b2[...] + x1.astype(jnp.float32))
        x = _layernorm(y, n2g, n2b)

    out_ref[...] = x


def _logits_kernel(x_ref, w_ref, b_ref, out_ref):
    out_ref[...] = (jnp.dot(x_ref[...], w_ref[...],
                            preferred_element_type=jnp.float32) + b_ref[...])


def kernel(word_emb, pos_emb, fc_out_w, fc_out_b, l0_sa_wq, l0_sa_wk, l0_sa_wv, l0_sa_wo, l0_sa_bo, l0_norm_g, l0_norm_b, l0_ca_wq, l0_ca_wk, l0_ca_wv, l0_ca_wo, l0_ca_bo, l0_norm1_g, l0_norm1_b, l0_norm2_g, l0_norm2_b, l0_ff_w1, l0_ff_b1, l0_ff_w2, l0_ff_b2, l1_sa_wq, l1_sa_wk, l1_sa_wv, l1_sa_wo, l1_sa_bo, l1_norm_g, l1_norm_b, l1_ca_wq, l1_ca_wk, l1_ca_wv, l1_ca_wo, l1_ca_bo, l1_norm1_g, l1_norm1_b, l1_norm2_g, l1_norm2_b, l1_ff_w1, l1_ff_b1, l1_ff_w2, l1_ff_b2, tokens, enc_out, src_mask):
    N, L = tokens.shape
    E = word_emb.shape[1]
    Vv = fc_out_w.shape[1]
    F = l0_ff_w1.shape[1]
    scale = E ** -0.5

    x = (word_emb[tokens] + pos_emb[jnp.arange(L)][None, :]).astype(jnp.bfloat16)
    enc = enc_out.astype(jnp.bfloat16)
    keep = src_mask[:, 0:1, :]

    def prep(wq, wk, wv, wo, bo, g, b):
        wq_s = (wq.astype(jnp.float32) * scale).astype(jnp.bfloat16)
        return [wq_s, wk, wv, wo, bo.reshape(1, E), g.reshape(1, E),
                b.reshape(1, E)]

    weights = []
    for lw in ((l0_sa_wq, l0_sa_wk, l0_sa_wv, l0_sa_wo, l0_sa_bo,
                l0_norm_g, l0_norm_b,
                l0_ca_wq, l0_ca_wk, l0_ca_wv, l0_ca_wo, l0_ca_bo,
                l0_norm1_g, l0_norm1_b,
                l0_ff_w1, l0_ff_b1, l0_ff_w2, l0_ff_b2,
                l0_norm2_g, l0_norm2_b),
               (l1_sa_wq, l1_sa_wk, l1_sa_wv, l1_sa_wo, l1_sa_bo,
                l1_norm_g, l1_norm_b,
                l1_ca_wq, l1_ca_wk, l1_ca_wv, l1_ca_wo, l1_ca_bo,
                l1_norm1_g, l1_norm1_b,
                l1_ff_w1, l1_ff_b1, l1_ff_w2, l1_ff_b2,
                l1_norm2_g, l1_norm2_b)):
        (sa_wq, sa_wk, sa_wv, sa_wo, sa_bo, ng, nb,
         ca_wq, ca_wk, ca_wv, ca_wo, ca_bo, n1g, n1b,
         w1, b1, w2, b2, n2g, n2b) = lw
        weights += prep(sa_wq, sa_wk, sa_wv, sa_wo, sa_bo, ng, nb)
        weights += prep(ca_wq, ca_wk, ca_wv, ca_wo, ca_bo, n1g, n1b)
        weights += [w1, b1.reshape(1, F), w2, b2.reshape(1, E),
                    n2g.reshape(1, E), n2b.reshape(1, E)]

    row = pl.BlockSpec((None, L, E), lambda b: (b, 0, 0))

    def wspec(w):
        return pl.BlockSpec(w.shape, lambda b: (0, 0))

    in_specs = [row, row, pl.BlockSpec((None, 1, L), lambda b: (b, 0, 0))]
    in_specs += [wspec(w) for w in weights]

    body = pl.pallas_call(
        _decoder_body_kernel,
        out_shape=jax.ShapeDtypeStruct((N, L, E), jnp.bfloat16),
        grid=(N,),
        in_specs=in_specs,
        out_specs=row,
        compiler_params=pltpu.CompilerParams(
            dimension_semantics=("parallel",),
            vmem_limit_bytes=56 * 1024 * 1024),
    )(x, enc, keep, *weights)

    tv = 2048 if Vv % 2048 == 0 else Vv
    logits = pl.pallas_call(
        _logits_kernel,
        out_shape=jax.ShapeDtypeStruct((N, L, Vv), jnp.float32),
        grid=(N, Vv // tv),
        in_specs=[pl.BlockSpec((None, L, E), lambda b, v: (b, 0, 0)),
                  pl.BlockSpec((E, tv), lambda b, v: (0, v)),
                  pl.BlockSpec((1, tv), lambda b, v: (0, v))],
        out_specs=pl.BlockSpec((None, L, tv), lambda b, v: (b, 0, v)),
        compiler_params=pltpu.CompilerParams(
            dimension_semantics=("parallel", "parallel"),
            vmem_limit_bytes=56 * 1024 * 1024),
    )(body, fc_out_w, fc_out_b.reshape(1, Vv))

    return logits


# speedup vs baseline: 2.3134x; 1.7605x over previous
import jax
import jax.numpy as jnp
from jax import lax
from jax.experimental import pallas as pl
from jax.experimental.pallas import tpu as pltpu

_HEADS = 8
_NEG = -1e30


def _layernorm(y, g_ref, b_ref):
    mean = jnp.mean(y, axis=-1, keepdims=True)
    var = jnp.mean((y - mean) ** 2, axis=-1, keepdims=True)
    return (((y - mean) * lax.rsqrt(var + 1e-5)) * g_ref[...]
            + b_ref[...]).astype(jnp.bfloat16)


def _attn_ln(xq, K, V, wq_ref, wo_ref, bo_ref, g_ref, b_ref, bias):
    Lq, E = xq.shape
    H = _HEADS
    D = E // H

    Q = jnp.dot(xq, wq_ref[...],
                preferred_element_type=jnp.float32).astype(jnp.bfloat16)
    Kt = K.T

    attn = None
    for h in range(H):
        sl = slice(h * D, (h + 1) * D)
        s = jnp.dot(Q[:, sl], Kt[sl, :],
                    preferred_element_type=jnp.float32)
        s = s + bias
        m = jnp.max(s, axis=-1, keepdims=True)
        p = jnp.exp2(s - m)
        linv = 1.0 / jnp.sum(p, axis=-1, keepdims=True)
        ctx = jnp.dot(p.astype(jnp.bfloat16), V[:, sl],
                      preferred_element_type=jnp.float32)
        ctx = (ctx * linv).astype(jnp.bfloat16)
        contrib = jnp.dot(ctx, wo_ref[sl, :],
                          preferred_element_type=jnp.float32)
        attn = contrib if attn is None else attn + contrib

    attn = attn + bo_ref[...]
    return _layernorm(attn + xq.astype(jnp.float32), g_ref, b_ref)


def _decoder_body_kernel(x_ref, enc_ref, keep_ref, *refs):
    out_ref = refs[-1]
    wrs = refs[:-1]
    num_layers = len(wrs) // 20
    nb_pair = x_ref.shape[0]

    xs = [x_ref[b] for b in range(nb_pair)]
    encs = [enc_ref[b] for b in range(nb_pair)]

    L = xs[0].shape[0]
    Lk = encs[0].shape[0]
    row = lax.broadcasted_iota(jnp.int32, (L, L), 0)
    col = lax.broadcasted_iota(jnp.int32, (L, L), 1)
    causal_bias = jnp.where(row >= col, 0.0, _NEG)
    cross_biases = [jnp.where(keep_ref[b, 0:1, :] != 0.0, 0.0, _NEG)
                    for b in range(nb_pair)]

    for li in range(num_layers):
        (sa_wq, sa_wk, sa_wv, sa_wo, sa_bo, ng, nb,
         ca_wq, ca_wk, ca_wv, ca_wo, ca_bo, n1g, n1b,
         w1, b1, w2, b2, n2g, n2b) = wrs[20 * li:20 * (li + 1)]

        for b in range(nb_pair):
            x, enc = xs[b], encs[b]

            K = jnp.dot(x, sa_wk[...],
                        preferred_element_type=jnp.float32).astype(jnp.bfloat16)
            V = jnp.dot(x, sa_wv[...],
                        preferred_element_type=jnp.float32).astype(jnp.bfloat16)
            q = _attn_ln(x, K, V, sa_wq, sa_wo, sa_bo, ng, nb, causal_bias)

            K2 = jnp.dot(enc, ca_wk[...],
                         preferred_element_type=jnp.float32).astype(jnp.bfloat16)
            V2 = jnp.dot(enc, ca_wv[...],
                         preferred_element_type=jnp.float32).astype(jnp.bfloat16)
            x1 = _attn_ln(q, K2, V2, ca_wq, ca_wo, ca_bo, n1g, n1b,
                          cross_biases[b])

            h = jnp.dot(x1, w1[...],
                        preferred_element_type=jnp.float32) + b1[...]
            h = jnp.maximum(h, 0.0).astype(jnp.bfloat16)
            y = (jnp.dot(h, w2[...], preferred_element_type=jnp.float32)
                 + b2[...] + x1.astype(jnp.float32))
            xs[b] = _layernorm(y, n2g, n2b)

    for b in range(nb_pair):
        out_ref[b] = xs[b]


def _logits_kernel(x_ref, w_ref, b_ref, out_ref):
    out_ref[...] = (jnp.dot(x_ref[...], w_ref[...],
                            preferred_element_type=jnp.float32) + b_ref[...])


def kernel(word_emb, pos_emb, fc_out_w, fc_out_b, l0_sa_wq, l0_sa_wk, l0_sa_wv, l0_sa_wo, l0_sa_bo, l0_norm_g, l0_norm_b, l0_ca_wq, l0_ca_wk, l0_ca_wv, l0_ca_wo, l0_ca_bo, l0_norm1_g, l0_norm1_b, l0_norm2_g, l0_norm2_b, l0_ff_w1, l0_ff_b1, l0_ff_w2, l0_ff_b2, l1_sa_wq, l1_sa_wk, l1_sa_wv, l1_sa_wo, l1_sa_bo, l1_norm_g, l1_norm_b, l1_ca_wq, l1_ca_wk, l1_ca_wv, l1_ca_wo, l1_ca_bo, l1_norm1_g, l1_norm1_b, l1_norm2_g, l1_norm2_b, l1_ff_w1, l1_ff_b1, l1_ff_w2, l1_ff_b2, tokens, enc_out, src_mask):
    N, L = tokens.shape
    E = word_emb.shape[1]
    Vv = fc_out_w.shape[1]
    F = l0_ff_w1.shape[1]
    scale = E ** -0.5

    x = (word_emb[tokens] + pos_emb[jnp.arange(L)][None, :]).astype(jnp.bfloat16)
    enc = enc_out.astype(jnp.bfloat16)
    keep = src_mask[:, 0:1, :]

    def prep(wq, wk, wv, wo, bo, g, b):
        wq_s = (wq.astype(jnp.float32) * (scale * 1.4426950408889634)
                ).astype(jnp.bfloat16)
        return [wq_s, wk, wv, wo, bo.reshape(1, E), g.reshape(1, E),
                b.reshape(1, E)]

    weights = []
    for lw in ((l0_sa_wq, l0_sa_wk, l0_sa_wv, l0_sa_wo, l0_sa_bo,
                l0_norm_g, l0_norm_b,
                l0_ca_wq, l0_ca_wk, l0_ca_wv, l0_ca_wo, l0_ca_bo,
                l0_norm1_g, l0_norm1_b,
                l0_ff_w1, l0_ff_b1, l0_ff_w2, l0_ff_b2,
                l0_norm2_g, l0_norm2_b),
               (l1_sa_wq, l1_sa_wk, l1_sa_wv, l1_sa_wo, l1_sa_bo,
                l1_norm_g, l1_norm_b,
                l1_ca_wq, l1_ca_wk, l1_ca_wv, l1_ca_wo, l1_ca_bo,
                l1_norm1_g, l1_norm1_b,
                l1_ff_w1, l1_ff_b1, l1_ff_w2, l1_ff_b2,
                l1_norm2_g, l1_norm2_b)):
        (sa_wq, sa_wk, sa_wv, sa_wo, sa_bo, ng, nb,
         ca_wq, ca_wk, ca_wv, ca_wo, ca_bo, n1g, n1b,
         w1, b1, w2, b2, n2g, n2b) = lw
        weights += prep(sa_wq, sa_wk, sa_wv, sa_wo, sa_bo, ng, nb)
        weights += prep(ca_wq, ca_wk, ca_wv, ca_wo, ca_bo, n1g, n1b)
        weights += [w1, b1.reshape(1, F), w2, b2.reshape(1, E),
                    n2g.reshape(1, E), n2b.reshape(1, E)]

    pair = 4 if N % 4 == 0 else (2 if N % 2 == 0 else 1)
    row = pl.BlockSpec((pair, L, E), lambda b: (b, 0, 0))

    def wspec(w):
        return pl.BlockSpec(w.shape, lambda b: (0, 0))

    in_specs = [row, row, pl.BlockSpec((pair, 1, L), lambda b: (b, 0, 0))]
    in_specs += [wspec(w) for w in weights]

    body = pl.pallas_call(
        _decoder_body_kernel,
        out_shape=jax.ShapeDtypeStruct((N, L, E), jnp.bfloat16),
        grid=(N // pair,),
        in_specs=in_specs,
        out_specs=row,
        compiler_params=pltpu.CompilerParams(
            dimension_semantics=("parallel",),
            vmem_limit_bytes=56 * 1024 * 1024),
    )(x, enc, keep, *weights)

    tv = 4096 if Vv % 4096 == 0 else Vv
    logits = pl.pallas_call(
        _logits_kernel,
        out_shape=jax.ShapeDtypeStruct((N, L, Vv), jnp.float32),
        grid=(Vv // tv, N),
        in_specs=[pl.BlockSpec((None, L, E), lambda v, b: (b, 0, 0)),
                  pl.BlockSpec((E, tv), lambda v, b: (0, v)),
                  pl.BlockSpec((1, tv), lambda v, b: (0, v))],
        out_specs=pl.BlockSpec((None, L, tv), lambda v, b: (b, 0, v)),
        compiler_params=pltpu.CompilerParams(
            dimension_semantics=("parallel", "parallel"),
            vmem_limit_bytes=56 * 1024 * 1024),
    )(body, fc_out_w, fc_out_b.reshape(1, Vv))

    return logits
